# Initial kernel scaffold; baseline (speedup 1.0000x reference)
import jax
import jax.numpy as jnp
from jax.experimental import pallas as pl


def kernel(psi, bank_vecs, w_rank, halt_w, halt_b, head_mix, out_scale):
    raise NotImplementedError("write your pallas kernel here")


# speedup vs baseline: 3.8939x; 3.8939x over previous
import functools

import jax
import jax.numpy as jnp
from jax.experimental import pallas as pl
from jax.experimental.pallas import tpu as pltpu

_TMAX = 4
_TEMP = 1.0
_THRESH = 0.99


def _core(pr_ref, pi_ref, brT_ref, biT_ref, br_ref, bi_ref,
          wrank_ref, hw_ref, hb_ref, osc_ref,
          outr_ref, outi_ref, ponder_ref,
          *, topk, rank):
    bn, dim = pr_ref.shape
    bank = brT_ref.shape[1]
    f32 = jnp.float32

    pr0 = pr_ref[...]
    pi0 = pi_ref[...]
    brT = brT_ref[...]
    biT = biT_ref[...]
    br = br_ref[...]
    bi = bi_ref[...]

    sr, si = pr0, pi0
    accr = jnp.zeros((bn, dim), f32)
    acci = jnp.zeros((bn, dim), f32)
    pcum = jnp.zeros((bn, 1), f32)
    psum = jnp.zeros((bn, 1), f32)
    iota = jax.lax.broadcasted_iota(jnp.int32, (bn, bank), 1).astype(f32)
    neg_inf = jnp.float32(-jnp.inf)

    for it in range(_TMAX):
        re = (jnp.dot(sr, brT, preferred_element_type=f32)
              + jnp.dot(si, biT, preferred_element_type=f32))
        im = (jnp.dot(si, brT, preferred_element_type=f32)
              - jnp.dot(sr, biT, preferred_element_type=f32))
        scores = re * re + im * im

        idxs, vals = [], []
        masked = scores
        for _ in range(topk):
            m = jnp.max(masked, axis=1, keepdims=True)
            idx = jnp.min(jnp.where(masked == m, iota, float(bank)),
                          axis=1, keepdims=True)
            idxs.append(idx)
            vals.append(m)
            masked = jnp.where(iota == idx, neg_inf, masked)

        es = [jnp.exp((v - vals[0]) / _TEMP) for v in vals]
        esum = functools.reduce(lambda a, b: a + b, es)
        wsel = [e / esum for e in es]

        Ur = [None] * rank
        Ui = [None] * rank
        for k in range(topk):
            oh = (iota == idxs[k]).astype(f32)
            fr = jnp.dot(oh, br, preferred_element_type=f32) * wsel[k]
            fi = jnp.dot(oh, bi, preferred_element_type=f32) * wsel[k]
            for r in range(rank):
                c = wrank_ref[k, r]
                if Ur[r] is None:
                    Ur[r] = c * fr
                    Ui[r] = c * fi
                else:
                    Ur[r] = Ur[r] + c * fr
                    Ui[r] = Ui[r] + c * fi

        prr = jnp.zeros((bn, dim), f32)
        pri = jnp.zeros((bn, dim), f32)
        alpha = jnp.zeros((bn, 1), f32)
        for r in range(rank):
            un = jnp.sqrt(jnp.sum(Ur[r] * Ur[r] + Ui[r] * Ui[r],
                                  axis=1, keepdims=True))
            inv = 1.0 / jnp.maximum(un, 1e-6)
            ur = Ur[r] * inv
            ui = Ui[r] * inv
            cr = jnp.sum(ur * sr + ui * si, axis=1, keepdims=True)
            ci = jnp.sum(ur * si - ui * sr, axis=1, keepdims=True)
            prr = prr + cr * ur - ci * ui
            pri = pri + cr * ui + ci * ur
            alpha = alpha + cr * cr + ci * ci

        sq = jnp.sum(prr * prr + pri * pri, axis=1, keepdims=True)
        inv2 = 1.0 / jnp.sqrt(jnp.maximum(sq, 1e-6))
        pnr = prr * inv2
        pni = pri * inv2

        beta = jnp.clip(1.0 - alpha, 0.0, 1.0)
        gr = jnp.sum(pnr * sr + pni * si, axis=1, keepdims=True)
        gi = jnp.sum(pnr * si - pni * sr, axis=1, keepdims=True)
        gamma = gr * gr + gi * gi

        l0 = alpha * hw_ref[0, 0] + beta * hw_ref[1, 0] + gamma * hw_ref[2, 0] + hb_ref[0, 0]
        l1 = alpha * hw_ref[0, 1] + beta * hw_ref[1, 1] + gamma * hw_ref[2, 1] + hb_ref[0, 1]
        l2 = alpha * hw_ref[0, 2] + beta * hw_ref[1, 2] + gamma * hw_ref[2, 2] + hb_ref[0, 2]
        mx = jnp.maximum(jnp.maximum(l0, l1), l2)
        e0 = jnp.exp(l0 - mx)
        e1 = jnp.exp(l1 - mx)
        e2 = jnp.exp(l2 - mx)
        ph = (e0 + e1) / (e0 + e1 + e2)

        still = (pcum < _THRESH).astype(f32)
        if it == _TMAX - 1:
            w = jnp.clip(1.0 - pcum, 0.0, 1.0)
        else:
            newc = pcum + ph * still
            over = (newc >= _THRESH).astype(f32)
            w = still * (over * jnp.clip(1.0 - pcum, 0.0, 1.0)
                         + (1.0 - over) * ph)
            pcum = still * (over * 1.0 + (1.0 - over) * newc) + (1.0 - still) * pcum

        accr = accr + w * pnr
        acci = acci + w * pni
        psum = psum + ph
        sr, si = pnr, pni

    scale = osc_ref[0, 0]
    outr_ref[...] = pr0 + scale * accr
    outi_ref[...] = pi0 + scale * acci
    ponder_ref[0, 0, 0] = jnp.sum(psum)


def kernel(psi, bank_vecs, w_rank, halt_w, halt_b, head_mix, out_scale):
    b, t, dim, _ = psi.shape
    bt = b * t
    bank = bank_vecs.shape[0]
    topk, rank = w_rank.shape
    f32 = jnp.float32

    psi_flat = psi.reshape(bt, dim, 2)
    pr = psi_flat[..., 0]
    pi = psi_flat[..., 1]
    br = bank_vecs[..., 0]
    bi = bank_vecs[..., 1]
    brT = br.T
    biT = bi.T
    hb2 = halt_b.reshape(1, 3).astype(f32)
    osc2 = out_scale.reshape(1, 1).astype(f32)

    bn = 256 if bt % 256 == 0 else bt
    g = bt // bn

    body = functools.partial(_core, topk=topk, rank=rank)
    outr, outi, ponder_parts = pl.pallas_call(
        body,
        grid=(g,),
        in_specs=[
            pl.BlockSpec((bn, dim), lambda i: (i, 0)),
            pl.BlockSpec((bn, dim), lambda i: (i, 0)),
            pl.BlockSpec((dim, bank), lambda i: (0, 0)),
            pl.BlockSpec((dim, bank), lambda i: (0, 0)),
            pl.BlockSpec((bank, dim), lambda i: (0, 0)),
            pl.BlockSpec((bank, dim), lambda i: (0, 0)),
            pl.BlockSpec(memory_space=pltpu.SMEM),
            pl.BlockSpec(memory_space=pltpu.SMEM),
            pl.BlockSpec(memory_space=pltpu.SMEM),
            pl.BlockSpec(memory_space=pltpu.SMEM),
        ],
        out_specs=[
            pl.BlockSpec((bn, dim), lambda i: (i, 0)),
            pl.BlockSpec((bn, dim), lambda i: (i, 0)),
            pl.BlockSpec((1, 1, 1), lambda i: (i, 0, 0), memory_space=pltpu.SMEM),
        ],
        out_shape=[
            jax.ShapeDtypeStruct((bt, dim), f32),
            jax.ShapeDtypeStruct((bt, dim), f32),
            jax.ShapeDtypeStruct((g, 1, 1), f32),
        ],
        compiler_params=pltpu.CompilerParams(
            dimension_semantics=("arbitrary",),
        ),
    )(pr, pi, brT, biT, br, bi, w_rank.astype(f32), halt_w.astype(f32), hb2, osc2)

    psi_out = jnp.stack([outr, outi], axis=-1).reshape(b, t, dim, 2)
    ponder = jnp.sum(ponder_parts) / jnp.float32(bt)
    return psi_out, ponder


# speedup vs baseline: 3.9114x; 1.0045x over previous
import functools

import jax
import jax.numpy as jnp
from jax.experimental import pallas as pl
from jax.experimental.pallas import tpu as pltpu

_TMAX = 4
_TEMP = 1.0
_THRESH = 0.99


def _core(pr_ref, pi_ref, brT_ref, biT_ref, bankc_ref,
          wrank_ref, hw_ref, hb_ref, osc_ref,
          outr_ref, outi_ref, ponder_ref,
          *, topk, rank):
    bn, dim = pr_ref.shape
    bank = brT_ref.shape[1]
    f32 = jnp.float32

    pr0 = pr_ref[...]
    pi0 = pi_ref[...]
    brT = brT_ref[...]
    biT = biT_ref[...]
    bankc = bankc_ref[...]

    sr, si = pr0, pi0
    accr = jnp.zeros((bn, dim), f32)
    acci = jnp.zeros((bn, dim), f32)
    pcum = jnp.zeros((bn, 1), f32)
    psum = jnp.zeros((bn, 1), f32)
    iota = jax.lax.broadcasted_iota(jnp.int32, (bn, bank), 1).astype(f32)
    neg_inf = jnp.float32(-jnp.inf)

    for it in range(_TMAX):
        re = (jnp.dot(sr, brT, preferred_element_type=f32)
              + jnp.dot(si, biT, preferred_element_type=f32))
        im = (jnp.dot(si, brT, preferred_element_type=f32)
              - jnp.dot(sr, biT, preferred_element_type=f32))
        scores = re * re + im * im

        idxs, vals = [], []
        masked = scores
        for _ in range(topk):
            m = jnp.max(masked, axis=1, keepdims=True)
            idx = jnp.min(jnp.where(masked == m, iota, float(bank)),
                          axis=1, keepdims=True)
            idxs.append(idx)
            vals.append(m)
            masked = jnp.where(iota == idx, neg_inf, masked)

        es = [jnp.exp((v - vals[0]) / _TEMP) for v in vals]
        esum = functools.reduce(lambda a, b: a + b, es)
        wsel = [e / esum for e in es]

        Ur = [None] * rank
        Ui = [None] * rank
        for k in range(topk):
            oh = (iota == idxs[k]).astype(jnp.bfloat16)
            fc = jnp.dot(oh, bankc, preferred_element_type=f32)
            fr = fc[:, :dim] * wsel[k]
            fi = fc[:, dim:] * wsel[k]
            for r in range(rank):
                c = wrank_ref[k, r]
                if Ur[r] is None:
                    Ur[r] = c * fr
                    Ui[r] = c * fi
                else:
                    Ur[r] = Ur[r] + c * fr
                    Ui[r] = Ui[r] + c * fi

        prr = jnp.zeros((bn, dim), f32)
        pri = jnp.zeros((bn, dim), f32)
        alpha = jnp.zeros((bn, 1), f32)
        for r in range(rank):
            un = jnp.sqrt(jnp.sum(Ur[r] * Ur[r] + Ui[r] * Ui[r],
                                  axis=1, keepdims=True))
            inv = 1.0 / jnp.maximum(un, 1e-6)
            ur = Ur[r] * inv
            ui = Ui[r] * inv
            cr = jnp.sum(ur * sr + ui * si, axis=1, keepdims=True)
            ci = jnp.sum(ur * si - ui * sr, axis=1, keepdims=True)
            prr = prr + cr * ur - ci * ui
            pri = pri + cr * ui + ci * ur
            alpha = alpha + cr * cr + ci * ci

        sq = jnp.sum(prr * prr + pri * pri, axis=1, keepdims=True)
        inv2 = 1.0 / jnp.sqrt(jnp.maximum(sq, 1e-6))
        pnr = prr * inv2
        pni = pri * inv2

        beta = jnp.clip(1.0 - alpha, 0.0, 1.0)
        gr = jnp.sum(pnr * sr + pni * si, axis=1, keepdims=True)
        gi = jnp.sum(pnr * si - pni * sr, axis=1, keepdims=True)
        gamma = gr * gr + gi * gi

        l0 = alpha * hw_ref[0, 0] + beta * hw_ref[1, 0] + gamma * hw_ref[2, 0] + hb_ref[0, 0]
        l1 = alpha * hw_ref[0, 1] + beta * hw_ref[1, 1] + gamma * hw_ref[2, 1] + hb_ref[0, 1]
        l2 = alpha * hw_ref[0, 2] + beta * hw_ref[1, 2] + gamma * hw_ref[2, 2] + hb_ref[0, 2]
        mx = jnp.maximum(jnp.maximum(l0, l1), l2)
        e0 = jnp.exp(l0 - mx)
        e1 = jnp.exp(l1 - mx)
        e2 = jnp.exp(l2 - mx)
        ph = (e0 + e1) / (e0 + e1 + e2)

        still = (pcum < _THRESH).astype(f32)
        if it == _TMAX - 1:
            w = jnp.clip(1.0 - pcum, 0.0, 1.0)
        else:
            newc = pcum + ph * still
            over = (newc >= _THRESH).astype(f32)
            w = still * (over * jnp.clip(1.0 - pcum, 0.0, 1.0)
                         + (1.0 - over) * ph)
            pcum = still * (over * 1.0 + (1.0 - over) * newc) + (1.0 - still) * pcum

        accr = accr + w * pnr
        acci = acci + w * pni
        psum = psum + ph
        sr, si = pnr, pni

    scale = osc_ref[0, 0]
    outr_ref[...] = pr0 + scale * accr
    outi_ref[...] = pi0 + scale * acci
    ponder_ref[0, 0, 0] = jnp.sum(psum)


def kernel(psi, bank_vecs, w_rank, halt_w, halt_b, head_mix, out_scale):
    b, t, dim, _ = psi.shape
    bt = b * t
    bank = bank_vecs.shape[0]
    topk, rank = w_rank.shape
    f32 = jnp.float32

    psi_flat = psi.reshape(bt, dim, 2)
    pr = psi_flat[..., 0]
    pi = psi_flat[..., 1]
    br = bank_vecs[..., 0]
    bi = bank_vecs[..., 1]
    brT = br.T
    biT = bi.T
    bankc = jnp.concatenate([br, bi], axis=1).astype(jnp.bfloat16)
    hb2 = halt_b.reshape(1, 3).astype(f32)
    osc2 = out_scale.reshape(1, 1).astype(f32)

    bn = 256 if bt % 256 == 0 else bt
    g = bt // bn

    body = functools.partial(_core, topk=topk, rank=rank)
    outr, outi, ponder_parts = pl.pallas_call(
        body,
        grid=(g,),
        in_specs=[
            pl.BlockSpec((bn, dim), lambda i: (i, 0)),
            pl.BlockSpec((bn, dim), lambda i: (i, 0)),
            pl.BlockSpec((dim, bank), lambda i: (0, 0)),
            pl.BlockSpec((dim, bank), lambda i: (0, 0)),
            pl.BlockSpec((bank, 2 * dim), lambda i: (0, 0)),
            pl.BlockSpec(memory_space=pltpu.SMEM),
            pl.BlockSpec(memory_space=pltpu.SMEM),
            pl.BlockSpec(memory_space=pltpu.SMEM),
            pl.BlockSpec(memory_space=pltpu.SMEM),
        ],
        out_specs=[
            pl.BlockSpec((bn, dim), lambda i: (i, 0)),
            pl.BlockSpec((bn, dim), lambda i: (i, 0)),
            pl.BlockSpec((1, 1, 1), lambda i: (i, 0, 0), memory_space=pltpu.SMEM),
        ],
        out_shape=[
            jax.ShapeDtypeStruct((bt, dim), f32),
            jax.ShapeDtypeStruct((bt, dim), f32),
            jax.ShapeDtypeStruct((g, 1, 1), f32),
        ],
        compiler_params=pltpu.CompilerParams(
            dimension_semantics=("arbitrary",),
            vmem_limit_bytes=100 * 1024 * 1024,
        ),
    )(pr, pi, brT, biT, bankc, w_rank.astype(f32), halt_w.astype(f32), hb2, osc2)

    psi_out = jnp.stack([outr, outi], axis=-1).reshape(b, t, dim, 2)
    ponder = jnp.sum(ponder_parts) / jnp.float32(bt)
    return psi_out, ponder


# speedup vs baseline: 5.8325x; 1.4912x over previous
import functools

import jax
import jax.numpy as jnp
from jax.experimental import pallas as pl
from jax.experimental.pallas import tpu as pltpu

_TMAX = 4
_TEMP = 1.0
_THRESH = 0.99


def _core(pr_ref, pi_ref, brT_ref, biT_ref, bankc_ref,
          wrankv_ref, hw_ref, hb_ref, osc_ref,
          outr_ref, outi_ref, ponder_ref,
          *, topk, rank):
    bn, dim = pr_ref.shape
    bank = brT_ref.shape[1]
    f32 = jnp.float32

    pr0 = pr_ref[...]
    pi0 = pi_ref[...]
    brT = brT_ref[...]
    biT = biT_ref[...]
    bankc = bankc_ref[...]

    sr, si = pr0, pi0
    accr = jnp.zeros((bn, dim), f32)
    acci = jnp.zeros((bn, dim), f32)
    pcum = jnp.zeros((bn, 1), f32)
    psum = jnp.zeros((bn, 1), f32)
    iota = jax.lax.broadcasted_iota(jnp.int32, (bn, bank), 1).astype(f32)
    neg_inf = jnp.float32(-jnp.inf)

    for it in range(_TMAX):
        re = (jnp.dot(sr, brT, preferred_element_type=f32)
              + jnp.dot(si, biT, preferred_element_type=f32))
        im = (jnp.dot(si, brT, preferred_element_type=f32)
              - jnp.dot(sr, biT, preferred_element_type=f32))
        scores = re * re + im * im

        idxs, vals = [], []
        masked = scores
        for _ in range(topk):
            m = jnp.max(masked, axis=1, keepdims=True)
            idx = jnp.min(jnp.where(masked == m, iota, float(bank)),
                          axis=1, keepdims=True)
            idxs.append(idx)
            vals.append(m)
            masked = jnp.where(iota == idx, neg_inf, masked)

        es = [jnp.exp((v - vals[0]) / _TEMP) for v in vals]
        esum = functools.reduce(lambda a, b: a + b, es)
        wsel = [e / esum for e in es]

        ohs, frs, fis, hrs, his = [], [], [], [], []
        for k in range(topk):
            ohf = (iota == idxs[k]).astype(f32)
            ohs.append(ohf)
            fc = jnp.dot(ohf.astype(jnp.bfloat16), bankc,
                         preferred_element_type=f32)
            frs.append(fc[:, :dim])
            fis.append(fc[:, dim:])
            hrs.append(jnp.sum(ohf * re, axis=1, keepdims=True))
            his.append(jnp.sum(ohf * im, axis=1, keepdims=True))

        gre = {}
        for k in range(topk):
            for l in range(k, topk):
                gre[(k, l)] = jnp.sum(frs[k] * frs[l] + fis[k] * fis[l],
                                      axis=1, keepdims=True)

        a = [wsel[k] * wrankv_ref[k:k + 1, :] for k in range(topk)]
        nu2 = jnp.zeros((bn, rank), f32)
        for k in range(topk):
            for l in range(k, topk):
                gkl = gre[(k, l)] if k == l else 2.0 * gre[(k, l)]
                nu2 = nu2 + (a[k] * a[l]) * gkl
        minv = 1.0 / jnp.maximum(jnp.sqrt(jnp.maximum(nu2, 0.0)), 1e-6)

        tr = jnp.zeros((bn, rank), f32)
        ti = jnp.zeros((bn, rank), f32)
        for k in range(topk):
            tr = tr + a[k] * hrs[k]
            ti = ti + a[k] * his[k]
        cr = tr * minv
        ci = ti * minv
        alpha = jnp.sum(cr * cr + ci * ci, axis=1, keepdims=True)

        dr = cr * minv
        di = ci * minv
        prr = jnp.zeros((bn, dim), f32)
        pri = jnp.zeros((bn, dim), f32)
        mrs, mis = [], []
        for k in range(topk):
            mr = jnp.sum(dr * a[k], axis=1, keepdims=True)
            mi = jnp.sum(di * a[k], axis=1, keepdims=True)
            mrs.append(mr)
            mis.append(mi)
            prr = prr + mr * frs[k] - mi * fis[k]
            pri = pri + mr * fis[k] + mi * frs[k]

        sq = jnp.sum(prr * prr + pri * pri, axis=1, keepdims=True)
        inv2 = 1.0 / jnp.sqrt(jnp.maximum(sq, 1e-6))
        pnr = prr * inv2
        pni = pri * inv2

        beta = jnp.clip(1.0 - alpha, 0.0, 1.0)
        gr0 = jnp.zeros((bn, 1), f32)
        gi0 = jnp.zeros((bn, 1), f32)
        for k in range(topk):
            gr0 = gr0 + mrs[k] * hrs[k] + mis[k] * his[k]
            gi0 = gi0 + mrs[k] * his[k] - mis[k] * hrs[k]
        gr = gr0 * inv2
        gi = gi0 * inv2
        gamma = gr * gr + gi * gi

        l0 = alpha * hw_ref[0, 0] + beta * hw_ref[1, 0] + gamma * hw_ref[2, 0] + hb_ref[0, 0]
        l1 = alpha * hw_ref[0, 1] + beta * hw_ref[1, 1] + gamma * hw_ref[2, 1] + hb_ref[0, 1]
        l2 = alpha * hw_ref[0, 2] + beta * hw_ref[1, 2] + gamma * hw_ref[2, 2] + hb_ref[0, 2]
        mx = jnp.maximum(jnp.maximum(l0, l1), l2)
        e0 = jnp.exp(l0 - mx)
        e1 = jnp.exp(l1 - mx)
        e2 = jnp.exp(l2 - mx)
        ph = (e0 + e1) / (e0 + e1 + e2)

        still = (pcum < _THRESH).astype(f32)
        if it == _TMAX - 1:
            w = jnp.clip(1.0 - pcum, 0.0, 1.0)
        else:
            newc = pcum + ph * still
            over = (newc >= _THRESH).astype(f32)
            w = still * (over * jnp.clip(1.0 - pcum, 0.0, 1.0)
                         + (1.0 - over) * ph)
            pcum = still * (over * 1.0 + (1.0 - over) * newc) + (1.0 - still) * pcum

        accr = accr + w * pnr
        acci = acci + w * pni
        psum = psum + ph
        sr, si = pnr, pni

    scale = osc_ref[0, 0]
    outr_ref[...] = pr0 + scale * accr
    outi_ref[...] = pi0 + scale * acci
    ponder_ref[0, 0, 0] = jnp.sum(psum)


def kernel(psi, bank_vecs, w_rank, halt_w, halt_b, head_mix, out_scale):
    b, t, dim, _ = psi.shape
    bt = b * t
    bank = bank_vecs.shape[0]
    topk, rank = w_rank.shape
    f32 = jnp.float32

    psi_flat = psi.reshape(bt, dim, 2)
    pr = psi_flat[..., 0]
    pi = psi_flat[..., 1]
    br = bank_vecs[..., 0]
    bi = bank_vecs[..., 1]
    brT = br.T
    biT = bi.T
    bankc = jnp.concatenate([br, bi], axis=1).astype(jnp.bfloat16)
    hb2 = halt_b.reshape(1, 3).astype(f32)
    osc2 = out_scale.reshape(1, 1).astype(f32)

    bn = 256 if bt % 256 == 0 else bt
    g = bt // bn

    body = functools.partial(_core, topk=topk, rank=rank)
    outr, outi, ponder_parts = pl.pallas_call(
        body,
        grid=(g,),
        in_specs=[
            pl.BlockSpec((bn, dim), lambda i: (i, 0)),
            pl.BlockSpec((bn, dim), lambda i: (i, 0)),
            pl.BlockSpec((dim, bank), lambda i: (0, 0)),
            pl.BlockSpec((dim, bank), lambda i: (0, 0)),
            pl.BlockSpec((bank, 2 * dim), lambda i: (0, 0)),
            pl.BlockSpec((topk, rank), lambda i: (0, 0)),
            pl.BlockSpec(memory_space=pltpu.SMEM),
            pl.BlockSpec(memory_space=pltpu.SMEM),
            pl.BlockSpec(memory_space=pltpu.SMEM),
        ],
        out_specs=[
            pl.BlockSpec((bn, dim), lambda i: (i, 0)),
            pl.BlockSpec((bn, dim), lambda i: (i, 0)),
            pl.BlockSpec((1, 1, 1), lambda i: (i, 0, 0), memory_space=pltpu.SMEM),
        ],
        out_shape=[
            jax.ShapeDtypeStruct((bt, dim), f32),
            jax.ShapeDtypeStruct((bt, dim), f32),
            jax.ShapeDtypeStruct((g, 1, 1), f32),
        ],
        compiler_params=pltpu.CompilerParams(
            dimension_semantics=("arbitrary",),
            vmem_limit_bytes=100 * 1024 * 1024,
        ),
    )(pr, pi, brT, biT, bankc, w_rank.astype(f32),
      halt_w.astype(f32), hb2, osc2)

    psi_out = jnp.stack([outr, outi], axis=-1).reshape(b, t, dim, 2)
    ponder = jnp.sum(ponder_parts) / jnp.float32(bt)
    return psi_out, ponder


# speedup vs baseline: 5.9773x; 1.0248x over previous
import functools

import jax
import jax.numpy as jnp
from jax.experimental import pallas as pl
from jax.experimental.pallas import tpu as pltpu

_TMAX = 4
_TEMP = 1.0
_THRESH = 0.99


def _core(pr_ref, pi_ref, brT_ref, biT_ref, bankc_ref,
          wrankv_ref, hw_ref, hb_ref, osc_ref,
          outr_ref, outi_ref, ponder_ref,
          *, topk, rank):
    bn, dim = pr_ref.shape
    bank = brT_ref.shape[1]
    f32 = jnp.float32

    pr0 = pr_ref[...]
    pi0 = pi_ref[...]
    brT = brT_ref[...]
    biT = biT_ref[...]
    bankc = bankc_ref[...]

    sr, si = pr0, pi0
    accr = jnp.zeros((bn, dim), f32)
    acci = jnp.zeros((bn, dim), f32)
    pcum = jnp.zeros((bn, 1), f32)
    psum = jnp.zeros((bn, 1), f32)
    neg_inf = jnp.float32(-jnp.inf)

    for it in range(_TMAX):
        re = (jnp.dot(sr, brT, preferred_element_type=f32)
              + jnp.dot(si, biT, preferred_element_type=f32))
        im = (jnp.dot(si, brT, preferred_element_type=f32)
              - jnp.dot(sr, biT, preferred_element_type=f32))
        scores = re * re + im * im

        vals, ohs = [], []
        masked = scores
        for _ in range(topk):
            m = jnp.max(masked, axis=1, keepdims=True)
            eq = masked == m
            ohs.append(eq.astype(f32))
            vals.append(m)
            masked = jnp.where(eq, neg_inf, masked)

        es = [jnp.exp((v - vals[0]) / _TEMP) for v in vals]
        esum = functools.reduce(lambda a, b: a + b, es)
        wsel = [e / esum for e in es]

        frs, fis, hrs, his = [], [], [], []
        for k in range(topk):
            ohf = ohs[k]
            fc = jnp.dot(ohf.astype(jnp.bfloat16), bankc,
                         preferred_element_type=f32)
            frs.append(fc[:, :dim])
            fis.append(fc[:, dim:])
            hrs.append(jnp.sum(ohf * re, axis=1, keepdims=True))
            his.append(jnp.sum(ohf * im, axis=1, keepdims=True))

        gre = {}
        for k in range(topk):
            for l in range(k, topk):
                gre[(k, l)] = jnp.sum(frs[k] * frs[l] + fis[k] * fis[l],
                                      axis=1, keepdims=True)

        a = [wsel[k] * wrankv_ref[k:k + 1, :] for k in range(topk)]
        nu2 = jnp.zeros((bn, rank), f32)
        for k in range(topk):
            for l in range(k, topk):
                gkl = gre[(k, l)] if k == l else 2.0 * gre[(k, l)]
                nu2 = nu2 + (a[k] * a[l]) * gkl
        minv = 1.0 / jnp.maximum(jnp.sqrt(jnp.maximum(nu2, 0.0)), 1e-6)

        tr = jnp.zeros((bn, rank), f32)
        ti = jnp.zeros((bn, rank), f32)
        for k in range(topk):
            tr = tr + a[k] * hrs[k]
            ti = ti + a[k] * his[k]
        cr = tr * minv
        ci = ti * minv
        alpha = jnp.sum(cr * cr + ci * ci, axis=1, keepdims=True)

        dr = cr * minv
        di = ci * minv
        prr = jnp.zeros((bn, dim), f32)
        pri = jnp.zeros((bn, dim), f32)
        mrs, mis = [], []
        for k in range(topk):
            mr = jnp.sum(dr * a[k], axis=1, keepdims=True)
            mi = jnp.sum(di * a[k], axis=1, keepdims=True)
            mrs.append(mr)
            mis.append(mi)
            prr = prr + mr * frs[k] - mi * fis[k]
            pri = pri + mr * fis[k] + mi * frs[k]

        sq = jnp.sum(prr * prr + pri * pri, axis=1, keepdims=True)
        inv2 = 1.0 / jnp.sqrt(jnp.maximum(sq, 1e-6))
        pnr = prr * inv2
        pni = pri * inv2

        beta = jnp.clip(1.0 - alpha, 0.0, 1.0)
        gr0 = jnp.zeros((bn, 1), f32)
        gi0 = jnp.zeros((bn, 1), f32)
        for k in range(topk):
            gr0 = gr0 + mrs[k] * hrs[k] + mis[k] * his[k]
            gi0 = gi0 + mrs[k] * his[k] - mis[k] * hrs[k]
        gr = gr0 * inv2
        gi = gi0 * inv2
        gamma = gr * gr + gi * gi

        l0 = alpha * hw_ref[0, 0] + beta * hw_ref[1, 0] + gamma * hw_ref[2, 0] + hb_ref[0, 0]
        l1 = alpha * hw_ref[0, 1] + beta * hw_ref[1, 1] + gamma * hw_ref[2, 1] + hb_ref[0, 1]
        l2 = alpha * hw_ref[0, 2] + beta * hw_ref[1, 2] + gamma * hw_ref[2, 2] + hb_ref[0, 2]
        mx = jnp.maximum(jnp.maximum(l0, l1), l2)
        e0 = jnp.exp(l0 - mx)
        e1 = jnp.exp(l1 - mx)
        e2 = jnp.exp(l2 - mx)
        ph = (e0 + e1) / (e0 + e1 + e2)

        still = (pcum < _THRESH).astype(f32)
        if it == _TMAX - 1:
            w = jnp.clip(1.0 - pcum, 0.0, 1.0)
        else:
            newc = pcum + ph * still
            over = (newc >= _THRESH).astype(f32)
            w = still * (over * jnp.clip(1.0 - pcum, 0.0, 1.0)
                         + (1.0 - over) * ph)
            pcum = still * (over * 1.0 + (1.0 - over) * newc) + (1.0 - still) * pcum

        accr = accr + w * pnr
        acci = acci + w * pni
        psum = psum + ph
        sr, si = pnr, pni

    scale = osc_ref[0, 0]
    outr_ref[...] = pr0 + scale * accr
    outi_ref[...] = pi0 + scale * acci
    ponder_ref[0, 0, 0] = jnp.sum(psum)


def kernel(psi, bank_vecs, w_rank, halt_w, halt_b, head_mix, out_scale):
    b, t, dim, _ = psi.shape
    bt = b * t
    bank = bank_vecs.shape[0]
    topk, rank = w_rank.shape
    f32 = jnp.float32

    psi_flat = psi.reshape(bt, dim, 2)
    pr = psi_flat[..., 0]
    pi = psi_flat[..., 1]
    br = bank_vecs[..., 0]
    bi = bank_vecs[..., 1]
    brT = br.T
    biT = bi.T
    bankc = jnp.concatenate([br, bi], axis=1).astype(jnp.bfloat16)
    hb2 = halt_b.reshape(1, 3).astype(f32)
    osc2 = out_scale.reshape(1, 1).astype(f32)

    bn = 256 if bt % 256 == 0 else bt
    g = bt // bn

    body = functools.partial(_core, topk=topk, rank=rank)
    outr, outi, ponder_parts = pl.pallas_call(
        body,
        grid=(g,),
        in_specs=[
            pl.BlockSpec((bn, dim), lambda i: (i, 0)),
            pl.BlockSpec((bn, dim), lambda i: (i, 0)),
            pl.BlockSpec((dim, bank), lambda i: (0, 0)),
            pl.BlockSpec((dim, bank), lambda i: (0, 0)),
            pl.BlockSpec((bank, 2 * dim), lambda i: (0, 0)),
            pl.BlockSpec((topk, rank), lambda i: (0, 0)),
            pl.BlockSpec(memory_space=pltpu.SMEM),
            pl.BlockSpec(memory_space=pltpu.SMEM),
            pl.BlockSpec(memory_space=pltpu.SMEM),
        ],
        out_specs=[
            pl.BlockSpec((bn, dim), lambda i: (i, 0)),
            pl.BlockSpec((bn, dim), lambda i: (i, 0)),
            pl.BlockSpec((1, 1, 1), lambda i: (i, 0, 0), memory_space=pltpu.SMEM),
        ],
        out_shape=[
            jax.ShapeDtypeStruct((bt, dim), f32),
            jax.ShapeDtypeStruct((bt, dim), f32),
            jax.ShapeDtypeStruct((g, 1, 1), f32),
        ],
        compiler_params=pltpu.CompilerParams(
            dimension_semantics=("arbitrary",),
            vmem_limit_bytes=100 * 1024 * 1024,
        ),
    )(pr, pi, brT, biT, bankc, w_rank.astype(f32),
      halt_w.astype(f32), hb2, osc2)

    psi_out = jnp.stack([outr, outi], axis=-1).reshape(b, t, dim, 2)
    ponder = jnp.sum(ponder_parts) / jnp.float32(bt)
    return psi_out, ponder
